# baseline (speedup 1.0000x reference)
import jax
import jax.numpy as jnp
from jax.experimental import pallas as pl
from jax.experimental.pallas import tpu as pltpu

_T, _D, _E, _K = 8192, 2048, 64, 8
_BT = 512
_NT = _T // _BT
_NBUF = 4


def _epilogue(logits):
    x = logits
    for _ in range(_K - 1):
        m = jnp.max(x, axis=-1, keepdims=True)
        x = jnp.where(x >= m, -jnp.inf, x)
    thr = jnp.max(x, axis=-1, keepdims=True)
    mask = logits >= thr
    rowmax = jnp.max(logits, axis=-1, keepdims=True)
    e = jnp.where(mask, jnp.exp(logits - rowmax), 0.0)
    probs = e / jnp.sum(e, axis=-1, keepdims=True)
    return mask.astype(jnp.int8), probs


def _router_stream(h_hbm, w_ref, mask_hbm, probs_hbm, logits_hbm,
                   buf, stg_mask, stg_probs, stg_logits, isem, osem):
    def in_copy(t, s):
        return pltpu.make_async_copy(
            h_hbm.at[pl.ds(t * _BT, _BT), :], buf.at[s], isem.at[s])

    def out_copies(t, s):
        return (
            pltpu.make_async_copy(
                stg_mask.at[s], mask_hbm.at[pl.ds(t * _BT, _BT), :],
                osem.at[s, 0]),
            pltpu.make_async_copy(
                stg_probs.at[s], probs_hbm.at[pl.ds(t * _BT, _BT), :],
                osem.at[s, 1]),
            pltpu.make_async_copy(
                stg_logits.at[s], logits_hbm.at[pl.ds(t * _BT, _BT), :],
                osem.at[s, 2]),
        )

    for j in range(_NBUF):
        in_copy(j, j).start()

    for t in range(_NT):
        s = t % _NBUF
        in_copy(t, s).wait()
        logits = jax.lax.dot_general(
            buf[s], w_ref[...],
            dimension_numbers=(((1,), (0,)), ((), ())),
            preferred_element_type=jnp.float32,
        )
        mask8, probs = _epilogue(logits)
        os = t % 2
        if t >= 2:
            for c in out_copies(t - 2, os):
                c.wait()
        stg_mask[os] = mask8
        stg_probs[os] = probs
        stg_logits[os] = logits
        for c in out_copies(t, os):
            c.start()
        nxt = t + _NBUF
        if nxt < _NT:
            in_copy(nxt, nxt % _NBUF).start()

    for t in (_NT - 2, _NT - 1):
        for c in out_copies(t, t % 2):
            c.wait()


@jax.jit
def kernel(h, W):
    t, d = h.shape
    e = W.shape[1]
    mask, probs, logits = pl.pallas_call(
        _router_stream,
        in_specs=[
            pl.BlockSpec(memory_space=pl.ANY),
            pl.BlockSpec((d, e), lambda: (0, 0)),
        ],
        out_specs=[
            pl.BlockSpec(memory_space=pl.ANY),
            pl.BlockSpec(memory_space=pl.ANY),
            pl.BlockSpec(memory_space=pl.ANY),
        ],
        out_shape=[
            jax.ShapeDtypeStruct((t, e), jnp.int8),
            jax.ShapeDtypeStruct((t, e), jnp.float32),
            jax.ShapeDtypeStruct((t, e), jnp.float32),
        ],
        scratch_shapes=[
            pltpu.VMEM((_NBUF, _BT, d), jnp.float32),
            pltpu.VMEM((2, _BT, e), jnp.int8),
            pltpu.VMEM((2, _BT, e), jnp.float32),
            pltpu.VMEM((2, _BT, e), jnp.float32),
            pltpu.SemaphoreType.DMA((_NBUF,)),
            pltpu.SemaphoreType.DMA((2, 3)),
        ],
    )(h, W)
    return (mask.astype(bool), probs, logits, logits)


# speedup vs baseline: 1.0197x; 1.0197x over previous
import jax
import jax.numpy as jnp
from jax.experimental import pallas as pl
from jax.experimental.pallas import tpu as pltpu

_T, _D, _E, _K = 8192, 2048, 64, 8
_BT = 2048
_NT = _T // _BT
_NBUF = 3


def _epilogue(logits):
    x = logits
    for _ in range(_K - 1):
        m = jnp.max(x, axis=-1, keepdims=True)
        x = jnp.where(x >= m, -jnp.inf, x)
    thr = jnp.max(x, axis=-1, keepdims=True)
    mask = logits >= thr
    rowmax = jnp.max(logits, axis=-1, keepdims=True)
    e = jnp.where(mask, jnp.exp(logits - rowmax), 0.0)
    probs = e / jnp.sum(e, axis=-1, keepdims=True)
    return mask.astype(jnp.int8), probs


def _router_stream(h_hbm, w_ref, mask_hbm, probs_hbm, logits_hbm,
                   buf, stg_mask, stg_probs, stg_logits, isem, osem):
    def in_copy(t, s):
        return pltpu.make_async_copy(
            h_hbm.at[pl.ds(t * _BT, _BT), :], buf.at[s], isem.at[s])

    def out_copies(t, s):
        return (
            pltpu.make_async_copy(
                stg_mask.at[s], mask_hbm.at[pl.ds(t * _BT, _BT), :],
                osem.at[s, 0]),
            pltpu.make_async_copy(
                stg_probs.at[s], probs_hbm.at[pl.ds(t * _BT, _BT), :],
                osem.at[s, 1]),
            pltpu.make_async_copy(
                stg_logits.at[s], logits_hbm.at[pl.ds(t * _BT, _BT), :],
                osem.at[s, 2]),
        )

    for j in range(_NBUF):
        in_copy(j, j).start()

    for t in range(_NT):
        s = t % _NBUF
        in_copy(t, s).wait()
        logits = jax.lax.dot_general(
            buf[s], w_ref[...],
            dimension_numbers=(((1,), (0,)), ((), ())),
            preferred_element_type=jnp.float32,
        )
        mask8, probs = _epilogue(logits)
        os = t % 2
        if t >= 2:
            for c in out_copies(t - 2, os):
                c.wait()
        stg_mask[os] = mask8
        stg_probs[os] = probs
        stg_logits[os] = logits
        for c in out_copies(t, os):
            c.start()
        nxt = t + _NBUF
        if nxt < _NT:
            in_copy(nxt, nxt % _NBUF).start()

    for t in (_NT - 2, _NT - 1):
        for c in out_copies(t, t % 2):
            c.wait()


@jax.jit
def kernel(h, W):
    t, d = h.shape
    e = W.shape[1]
    mask, probs, logits = pl.pallas_call(
        _router_stream,
        in_specs=[
            pl.BlockSpec(memory_space=pl.ANY),
            pl.BlockSpec((d, e), lambda: (0, 0)),
        ],
        out_specs=[
            pl.BlockSpec(memory_space=pl.ANY),
            pl.BlockSpec(memory_space=pl.ANY),
            pl.BlockSpec(memory_space=pl.ANY),
        ],
        out_shape=[
            jax.ShapeDtypeStruct((t, e), jnp.int8),
            jax.ShapeDtypeStruct((t, e), jnp.float32),
            jax.ShapeDtypeStruct((t, e), jnp.float32),
        ],
        scratch_shapes=[
            pltpu.VMEM((_NBUF, _BT, d), jnp.float32),
            pltpu.VMEM((2, _BT, e), jnp.int8),
            pltpu.VMEM((2, _BT, e), jnp.float32),
            pltpu.VMEM((2, _BT, e), jnp.float32),
            pltpu.SemaphoreType.DMA((_NBUF,)),
            pltpu.SemaphoreType.DMA((2, 3)),
        ],
    )(h, W)
    return (mask.astype(bool), probs, logits, logits)


# speedup vs baseline: 1.1339x; 1.1120x over previous
import functools

import jax
import jax.numpy as jnp
from jax.experimental import pallas as pl
from jax.experimental.pallas import tpu as pltpu

_T, _D, _E, _K = 8192, 2048, 64, 8
_BT = 2048


def _router_tile(h_ref, w_ref, mask_ref, probs_ref, logits_ref):
    logits = jax.lax.dot_general(
        h_ref[...], w_ref[...],
        dimension_numbers=(((1,), (0,)), ((), ())),
        preferred_element_type=jnp.float32,
    )
    x = logits
    for _ in range(_K - 1):
        m = jnp.max(x, axis=-1, keepdims=True)
        x = jnp.where(x >= m, -jnp.inf, x)
    thr = jnp.max(x, axis=-1, keepdims=True)
    mask = logits >= thr
    rowmax = jnp.max(logits, axis=-1, keepdims=True)
    e = jnp.where(mask, jnp.exp(logits - rowmax), 0.0)
    probs = e / jnp.sum(e, axis=-1, keepdims=True)
    mask_ref[...] = mask
    probs_ref[...] = probs
    logits_ref[...] = logits


@jax.jit
def kernel(h, W):
    t, d = h.shape
    e = W.shape[1]
    grid = (t // _BT,)
    mask, probs, logits = pl.pallas_call(
        _router_tile,
        grid=grid,
        in_specs=[
            pl.BlockSpec((_BT, d), lambda i: (i, 0)),
            pl.BlockSpec((d, e), lambda i: (0, 0)),
        ],
        out_specs=[
            pl.BlockSpec((_BT, e), lambda i: (i, 0)),
            pl.BlockSpec((_BT, e), lambda i: (i, 0)),
            pl.BlockSpec((_BT, e), lambda i: (i, 0)),
        ],
        out_shape=[
            jax.ShapeDtypeStruct((t, e), jnp.bool_),
            jax.ShapeDtypeStruct((t, e), jnp.float32),
            jax.ShapeDtypeStruct((t, e), jnp.float32),
        ],
    )(h, W)
    return (mask, probs, logits, logits)


# speedup vs baseline: 1.1722x; 1.0337x over previous
import functools

import jax
import jax.numpy as jnp
from jax.experimental import pallas as pl
from jax.experimental.pallas import tpu as pltpu

_T, _D, _E, _K = 8192, 2048, 64, 8
_BT = 2048


def _router_tile(h_ref, w_ref, mask_ref, probs_ref, logits_ref):
    logits = jax.lax.dot_general(
        h_ref[...], w_ref[...],
        dimension_numbers=(((1,), (0,)), ((), ())),
        preferred_element_type=jnp.float32,
    )
    rowmax = jnp.max(logits, axis=-1, keepdims=True)
    e_full = jnp.exp(logits - rowmax)
    x = jnp.where(logits >= rowmax, -jnp.inf, logits)
    for _ in range(_K - 2):
        m = jnp.max(x, axis=-1, keepdims=True)
        x = jnp.where(x >= m, -jnp.inf, x)
    thr = jnp.max(x, axis=-1, keepdims=True)
    mask = logits >= thr
    e = jnp.where(mask, e_full, 0.0)
    probs = e / jnp.sum(e, axis=-1, keepdims=True)
    mask_ref[...] = mask.astype(jnp.int8)
    probs_ref[...] = probs
    logits_ref[...] = logits


@jax.jit
def kernel(h, W):
    t, d = h.shape
    e = W.shape[1]
    grid = (t // _BT,)
    mask, probs, logits = pl.pallas_call(
        _router_tile,
        grid=grid,
        in_specs=[
            pl.BlockSpec((_BT, d), lambda i: (i, 0)),
            pl.BlockSpec((d, e), lambda i: (0, 0)),
        ],
        out_specs=[
            pl.BlockSpec((_BT, e), lambda i: (i, 0)),
            pl.BlockSpec((_BT, e), lambda i: (i, 0)),
            pl.BlockSpec((_BT, e), lambda i: (i, 0)),
        ],
        out_shape=[
            jax.ShapeDtypeStruct((t, e), jnp.int8),
            jax.ShapeDtypeStruct((t, e), jnp.float32),
            jax.ShapeDtypeStruct((t, e), jnp.float32),
        ],
    )(h, W)
    return (mask.astype(bool), probs, logits, logits)
